# baseline (speedup 1.0000x reference)
import functools

import jax
import jax.numpy as jnp
from jax import lax
from jax.experimental import pallas as pl
from jax.experimental.pallas import tpu as pltpu
from jax.experimental.pallas import tpu_sc as plsc

N_NODES = 10000
N_EDGES = 160000
D_FEAT = 256
HIDDEN = 256
HALF = 128
N_SEL = 10240

N_TILES = 16
EPT = N_EDGES // N_TILES
EK = 64
NCH_E = EPT // EK
ETAIL = EPT - NCH_E * EK
NG = 4
NI = 6
UNROLL = 12
RPT = N_SEL // N_TILES
RK = 32
NCH_R = RPT // RK
INIT_ROWS = 624
INIT_TAIL = N_NODES - N_TILES * INIT_ROWS
MTILE = 1000


def _mm_body(x_ref, w_ref, b_ref, o0_ref, o1_ref, o2_ref, o3_ref):
    x = x_ref[...]
    dn = (((1,), (1,)), ((), ()))
    h = lax.dot_general(x, w_ref[:, :D_FEAT], dn,
                        preferred_element_type=jnp.float32) + b_ref[...]
    g = lax.dot_general(x, w_ref[:, D_FEAT:], dn,
                        preferred_element_type=jnp.float32)
    o0_ref[...] = h[:, :HALF]
    o1_ref[...] = h[:, HALF:]
    o2_ref[...] = g[:, :HALF]
    o3_ref[...] = g[:, HALF:]


def _tc_matmul(features, w, b):
    part = pl.BlockSpec((MTILE, HALF), lambda m: (m, 0))
    pshape = jax.ShapeDtypeStruct((N_NODES, HALF), jnp.float32)
    return pl.pallas_call(
        _mm_body,
        grid=(N_NODES // MTILE,),
        in_specs=[
            pl.BlockSpec((MTILE, D_FEAT), lambda m: (m, 0)),
            pl.BlockSpec((HIDDEN, 2 * D_FEAT), lambda m: (0, 0)),
            pl.BlockSpec((1, HIDDEN), lambda m: (0, 0)),
        ],
        out_specs=[part, part, part, part],
        out_shape=[pshape, pshape, pshape, pshape],
    )(features.astype(jnp.bfloat16), w.astype(jnp.bfloat16),
      b.reshape(1, HIDDEN))


_SC_MESH = plsc.VectorSubcoreMesh(core_axis_name="c", subcore_axis_name="s")


@functools.partial(
    pl.kernel,
    out_type=jax.ShapeDtypeStruct((N_SEL, HIDDEN), jnp.float32),
    mesh=_SC_MESH,
    scratch_types=[
        pltpu.VMEM_SHARED((N_NODES, HALF), jnp.float32),
        *([pltpu.VMEM((EK,), jnp.int32)] * NI),
        *([pltpu.VMEM((EK,), jnp.int32)] * NI),
        pltpu.VMEM((ETAIL,), jnp.int32),
        pltpu.VMEM((ETAIL,), jnp.int32),
        pltpu.VMEM((NG, EK, HALF), jnp.float32),
        pltpu.VMEM((RK,), jnp.int32),
        pltpu.VMEM((RK,), jnp.int32),
        pltpu.VMEM((2, RK, HALF), jnp.float32),
        pltpu.SemaphoreType.DMA((NI,)),
        pltpu.SemaphoreType.DMA((NI,)),
        pltpu.SemaphoreType.DMA((NG,)),
        pltpu.SemaphoreType.DMA((NG,)),
        pltpu.SemaphoreType.DMA((2,)),
        pltpu.SemaphoreType.DMA((2,)),
        pltpu.SemaphoreType.DMA((2,)),
    ],
)
def _sc_aggregate(dsts, srcs, h0, h1, g0, g1, flat, out, acc,
                  dx0, dx1, dx2, dx3, dx4, dx5,
                  sx0, sx1, sx2, sx3, sx4, sx5, dtail,
                  stail, grows, fx0, fx1, obuf, idsem, issem, gsem,
                  ssem, fsem, g2sem, osem):
    cc = lax.axis_index("c")
    s = lax.axis_index("s")
    dix = (dx0, dx1, dx2, dx3, dx4, dx5)
    six = (sx0, sx1, sx2, sx3, sx4, sx5)
    fx = (fx0, fx1)
    ebase = s * EPT

    def load_eidx(c, r):
        pltpu.async_copy(dsts.at[pl.ds(ebase + c * EK, EK)], dix[r],
                         idsem.at[r])
        pltpu.async_copy(srcs.at[pl.ds(ebase + c * EK, EK)], six[r],
                         issem.at[r])

    def wait_eidx(r):
        pltpu.make_async_copy(dsts.at[pl.ds(0, EK)], dix[r],
                              idsem.at[r]).wait()
        pltpu.make_async_copy(srcs.at[pl.ds(0, EK)], six[r],
                              issem.at[r]).wait()

    def gather(r, q):
        @pl.when(cc == 0)
        def _():
            pltpu.async_copy(g0.at[six[r]], grows.at[q], gsem.at[q])

        @pl.when(cc == 1)
        def _():
            pltpu.async_copy(g1.at[six[r]], grows.at[q], gsem.at[q])

    def wait_gather(r, q):
        pltpu.make_async_copy(g0.at[six[r]], grows.at[q],
                              gsem.at[q]).wait()

    def scatter(r, q):
        pltpu.async_copy(grows.at[q], acc.at[dix[r]], ssem.at[q], add=True)

    def wait_scatter(r, q):
        pltpu.make_async_copy(grows.at[q], acc.at[dix[r]],
                              ssem.at[q]).wait()

    for k in range(NI):
        load_eidx(k, k)
    pltpu.async_copy(flat.at[pl.ds(s * RPT, RK)], fx0, fsem.at[0])

    r0 = s * INIT_ROWS

    @pl.when(cc == 0)
    def _():
        pltpu.sync_copy(h0.at[pl.ds(r0, INIT_ROWS)],
                        acc.at[pl.ds(r0, INIT_ROWS)])

    @pl.when(cc == 1)
    def _():
        pltpu.sync_copy(h1.at[pl.ds(r0, INIT_ROWS)],
                        acc.at[pl.ds(r0, INIT_ROWS)])

    tail0 = N_TILES * INIT_ROWS

    @pl.when(jnp.logical_and(s == N_TILES - 1, cc == 0))
    def _():
        pltpu.sync_copy(h0.at[pl.ds(tail0, INIT_TAIL)],
                        acc.at[pl.ds(tail0, INIT_TAIL)])

    @pl.when(jnp.logical_and(s == N_TILES - 1, cc == 1))
    def _():
        pltpu.sync_copy(h1.at[pl.ds(tail0, INIT_TAIL)],
                        acc.at[pl.ds(tail0, INIT_TAIL)])

    for k in range(3):
        wait_eidx(k)
        gather(k, k)
    plsc.subcore_barrier()

    def block(t, carry):
        for k in range(UNROLL):
            c = t * UNROLL + k
            q, r = k % NG, k % NI
            wait_gather(r, q)
            scatter(r, q)

            @pl.when(c >= 1)
            def _():
                wait_scatter((k - 1) % NI, (k - 1) % NG)

            @pl.when(jnp.logical_and(c >= 1, c + 5 < NCH_E))
            def _():
                load_eidx(c + 5, (k + 5) % NI)

            @pl.when(c + 3 < NCH_E)
            def _():
                wait_eidx((k + 3) % NI)
                gather((k + 3) % NI, (k + 3) % NG)

        return carry

    lax.fori_loop(0, NCH_E // UNROLL, block, 0)
    wait_scatter((NCH_E - 1) % NI, (NCH_E - 1) % NG)

    pltpu.sync_copy(dsts.at[pl.ds(ebase + NCH_E * EK, ETAIL)], dtail)
    pltpu.sync_copy(srcs.at[pl.ds(ebase + NCH_E * EK, ETAIL)], stail)
    tgt = grows.at[0, pl.ds(0, ETAIL)]

    @pl.when(cc == 0)
    def _():
        pltpu.async_copy(g0.at[stail], tgt, gsem.at[0]).wait()

    @pl.when(cc == 1)
    def _():
        pltpu.async_copy(g1.at[stail], tgt, gsem.at[0]).wait()

    pltpu.sync_copy(tgt, acc.at[dtail], add=True)
    plsc.subcore_barrier()

    fbase = s * RPT

    def load_fidx(i, sl):
        pltpu.async_copy(flat.at[pl.ds(fbase + i * RK, RK)], fx[sl],
                         fsem.at[sl])

    def wait_fidx(sl):
        pltpu.make_async_copy(flat.at[pl.ds(0, RK)], fx[sl],
                              fsem.at[sl]).wait()

    def gather2(sl):
        pltpu.async_copy(acc.at[fx[sl]], obuf.at[sl], g2sem.at[sl])

    def wait_gather2(sl):
        pltpu.make_async_copy(acc.at[fx[sl]], obuf.at[sl],
                              g2sem.at[sl]).wait()

    def store2(i, sl):
        b0 = fbase + i * RK

        @pl.when(cc == 0)
        def _():
            pltpu.async_copy(obuf.at[sl],
                             out.at[pl.ds(b0, RK), pl.ds(0, HALF)],
                             osem.at[sl])

        @pl.when(cc == 1)
        def _():
            pltpu.async_copy(obuf.at[sl],
                             out.at[pl.ds(b0, RK), pl.ds(HALF, HALF)],
                             osem.at[sl])

    def store_wait(i, sl):
        b0 = fbase + i * RK
        pltpu.make_async_copy(obuf.at[sl],
                              out.at[pl.ds(b0, RK), pl.ds(0, HALF)],
                              osem.at[sl]).wait()

    wait_fidx(0)
    load_fidx(1, 1)
    gather2(0)
    for i in range(NCH_R):
        sl = i % 2
        wait_gather2(sl)
        if i + 2 < NCH_R:
            load_fidx(i + 2, sl)
        if i + 1 < NCH_R:
            if i >= 1:
                store_wait(i - 1, 1 - sl)
            wait_fidx(1 - sl)
            gather2(1 - sl)

        @plsc.parallel_loop(0, RK, unroll=2)
        def _(r):
            for j in range(HALF // 16):
                v = obuf[sl, r, pl.ds(j * 16, 16)]
                obuf[sl, r, pl.ds(j * 16, 16)] = jnp.maximum(v, v * 0.01)

        store2(i, sl)
    store_wait(NCH_R - 2, 0)
    store_wait(NCH_R - 1, 1)


def kernel(node_x, edge_index, features_n_f, W, b):
    flat = node_x.reshape(-1).astype(jnp.int32)
    h0, h1, g0, g1 = _tc_matmul(features_n_f, W, b)
    out = _sc_aggregate(edge_index[0].astype(jnp.int32),
                        edge_index[1].astype(jnp.int32), h0, h1, g0, g1,
                        flat)
    return out.reshape(node_x.shape[0], node_x.shape[1], HIDDEN)


# speedup vs baseline: 1.0311x; 1.0311x over previous
import functools

import jax
import jax.numpy as jnp
from jax import lax
from jax.experimental import pallas as pl
from jax.experimental.pallas import tpu as pltpu
from jax.experimental.pallas import tpu_sc as plsc

N_NODES = 10000
N_EDGES = 160000
D_FEAT = 256
HIDDEN = 256
HALF = 128
N_SEL = 10240

N_TILES = 16
EPT = N_EDGES // N_TILES
EK = 64
NCH_E = EPT // EK
ETAIL = EPT - NCH_E * EK
NG = 4
NI = 6
UNROLL = 12
RPT = N_SEL // N_TILES
RK = 32
NCH_R = RPT // RK
INIT_ROWS = 624
INIT_TAIL = N_NODES - N_TILES * INIT_ROWS
MTILE = 1000


def _mm_body(x_ref, w_ref, b_ref, o0_ref, o1_ref, o2_ref, o3_ref):
    x = x_ref[...]
    dn = (((1,), (1,)), ((), ()))
    h = lax.dot_general(x, w_ref[:, :D_FEAT], dn,
                        preferred_element_type=jnp.float32) + b_ref[...]
    g = lax.dot_general(x, w_ref[:, D_FEAT:], dn,
                        preferred_element_type=jnp.float32)
    o0_ref[...] = h[:, :HALF]
    o1_ref[...] = h[:, HALF:]
    o2_ref[...] = g[:, :HALF]
    o3_ref[...] = g[:, HALF:]


def _tc_matmul(features, w, b):
    part = pl.BlockSpec((MTILE, HALF), lambda m: (m, 0))
    pshape = jax.ShapeDtypeStruct((N_NODES, HALF), jnp.float32)
    return pl.pallas_call(
        _mm_body,
        grid=(N_NODES // MTILE,),
        in_specs=[
            pl.BlockSpec((MTILE, D_FEAT), lambda m: (m, 0)),
            pl.BlockSpec((HIDDEN, 2 * D_FEAT), lambda m: (0, 0)),
            pl.BlockSpec((1, HIDDEN), lambda m: (0, 0)),
        ],
        out_specs=[part, part, part, part],
        out_shape=[pshape, pshape, pshape, pshape],
    )(features, w, b.reshape(1, HIDDEN))


_SC_MESH = plsc.VectorSubcoreMesh(core_axis_name="c", subcore_axis_name="s")


@functools.partial(
    pl.kernel,
    out_type=jax.ShapeDtypeStruct((N_SEL, HIDDEN), jnp.float32),
    mesh=_SC_MESH,
    scratch_types=[
        pltpu.VMEM_SHARED((N_NODES, HALF), jnp.float32),
        *([pltpu.VMEM((EK,), jnp.int32)] * NI),
        *([pltpu.VMEM((EK,), jnp.int32)] * NI),
        pltpu.VMEM((ETAIL,), jnp.int32),
        pltpu.VMEM((ETAIL,), jnp.int32),
        pltpu.VMEM((NG, EK, HALF), jnp.float32),
        pltpu.VMEM((RK,), jnp.int32),
        pltpu.VMEM((RK,), jnp.int32),
        pltpu.VMEM((2, RK, HALF), jnp.float32),
        pltpu.SemaphoreType.DMA((NI,)),
        pltpu.SemaphoreType.DMA((NI,)),
        pltpu.SemaphoreType.DMA((NG,)),
        pltpu.SemaphoreType.DMA((NG,)),
        pltpu.SemaphoreType.DMA((2,)),
        pltpu.SemaphoreType.DMA((2,)),
        pltpu.SemaphoreType.DMA((2,)),
    ],
)
def _sc_aggregate(dsts, srcs, h0, h1, g0, g1, flat, out, acc,
                  dx0, dx1, dx2, dx3, dx4, dx5,
                  sx0, sx1, sx2, sx3, sx4, sx5, dtail,
                  stail, grows, fx0, fx1, obuf, idsem, issem, gsem,
                  ssem, fsem, g2sem, osem):
    cc = lax.axis_index("c")
    s = lax.axis_index("s")
    dix = (dx0, dx1, dx2, dx3, dx4, dx5)
    six = (sx0, sx1, sx2, sx3, sx4, sx5)
    fx = (fx0, fx1)
    ebase = s * EPT

    def load_eidx(c, r):
        pltpu.async_copy(dsts.at[pl.ds(ebase + c * EK, EK)], dix[r],
                         idsem.at[r])
        pltpu.async_copy(srcs.at[pl.ds(ebase + c * EK, EK)], six[r],
                         issem.at[r])

    def wait_eidx(r):
        pltpu.make_async_copy(dsts.at[pl.ds(0, EK)], dix[r],
                              idsem.at[r]).wait()
        pltpu.make_async_copy(srcs.at[pl.ds(0, EK)], six[r],
                              issem.at[r]).wait()

    def gather(r, q):
        @pl.when(cc == 0)
        def _():
            pltpu.async_copy(g0.at[six[r]], grows.at[q], gsem.at[q])

        @pl.when(cc == 1)
        def _():
            pltpu.async_copy(g1.at[six[r]], grows.at[q], gsem.at[q])

    def wait_gather(r, q):
        pltpu.make_async_copy(g0.at[six[r]], grows.at[q],
                              gsem.at[q]).wait()

    def scatter(r, q):
        pltpu.async_copy(grows.at[q], acc.at[dix[r]], ssem.at[q], add=True)

    def wait_scatter(r, q):
        pltpu.make_async_copy(grows.at[q], acc.at[dix[r]],
                              ssem.at[q]).wait()

    for k in range(NI):
        load_eidx(k, k)
    pltpu.async_copy(flat.at[pl.ds(s * RPT, RK)], fx0, fsem.at[0])

    r0 = s * INIT_ROWS

    @pl.when(cc == 0)
    def _():
        pltpu.sync_copy(h0.at[pl.ds(r0, INIT_ROWS)],
                        acc.at[pl.ds(r0, INIT_ROWS)])

    @pl.when(cc == 1)
    def _():
        pltpu.sync_copy(h1.at[pl.ds(r0, INIT_ROWS)],
                        acc.at[pl.ds(r0, INIT_ROWS)])

    tail0 = N_TILES * INIT_ROWS

    @pl.when(jnp.logical_and(s == N_TILES - 1, cc == 0))
    def _():
        pltpu.sync_copy(h0.at[pl.ds(tail0, INIT_TAIL)],
                        acc.at[pl.ds(tail0, INIT_TAIL)])

    @pl.when(jnp.logical_and(s == N_TILES - 1, cc == 1))
    def _():
        pltpu.sync_copy(h1.at[pl.ds(tail0, INIT_TAIL)],
                        acc.at[pl.ds(tail0, INIT_TAIL)])

    for k in range(3):
        wait_eidx(k)
        gather(k, k)
    plsc.subcore_barrier()

    def block(t, carry):
        for k in range(UNROLL):
            c = t * UNROLL + k
            q, r = k % NG, k % NI
            wait_gather(r, q)
            scatter(r, q)

            @pl.when(c >= 1)
            def _():
                wait_scatter((k - 1) % NI, (k - 1) % NG)

            @pl.when(jnp.logical_and(c >= 1, c + 5 < NCH_E))
            def _():
                load_eidx(c + 5, (k + 5) % NI)

            @pl.when(c + 3 < NCH_E)
            def _():
                wait_eidx((k + 3) % NI)
                gather((k + 3) % NI, (k + 3) % NG)

        return carry

    lax.fori_loop(0, NCH_E // UNROLL, block, 0)
    wait_scatter((NCH_E - 1) % NI, (NCH_E - 1) % NG)

    pltpu.sync_copy(dsts.at[pl.ds(ebase + NCH_E * EK, ETAIL)], dtail)
    pltpu.sync_copy(srcs.at[pl.ds(ebase + NCH_E * EK, ETAIL)], stail)
    tgt = grows.at[0, pl.ds(0, ETAIL)]

    @pl.when(cc == 0)
    def _():
        pltpu.async_copy(g0.at[stail], tgt, gsem.at[0]).wait()

    @pl.when(cc == 1)
    def _():
        pltpu.async_copy(g1.at[stail], tgt, gsem.at[0]).wait()

    pltpu.sync_copy(tgt, acc.at[dtail], add=True)
    plsc.subcore_barrier()

    fbase = s * RPT

    def load_fidx(i, sl):
        pltpu.async_copy(flat.at[pl.ds(fbase + i * RK, RK)], fx[sl],
                         fsem.at[sl])

    def wait_fidx(sl):
        pltpu.make_async_copy(flat.at[pl.ds(0, RK)], fx[sl],
                              fsem.at[sl]).wait()

    def gather2(sl):
        pltpu.async_copy(acc.at[fx[sl]], obuf.at[sl], g2sem.at[sl])

    def wait_gather2(sl):
        pltpu.make_async_copy(acc.at[fx[sl]], obuf.at[sl],
                              g2sem.at[sl]).wait()

    def store2(i, sl):
        b0 = fbase + i * RK

        @pl.when(cc == 0)
        def _():
            pltpu.async_copy(obuf.at[sl],
                             out.at[pl.ds(b0, RK), pl.ds(0, HALF)],
                             osem.at[sl])

        @pl.when(cc == 1)
        def _():
            pltpu.async_copy(obuf.at[sl],
                             out.at[pl.ds(b0, RK), pl.ds(HALF, HALF)],
                             osem.at[sl])

    def store_wait(i, sl):
        b0 = fbase + i * RK
        pltpu.make_async_copy(obuf.at[sl],
                              out.at[pl.ds(b0, RK), pl.ds(0, HALF)],
                              osem.at[sl]).wait()

    wait_fidx(0)
    load_fidx(1, 1)
    gather2(0)
    for i in range(NCH_R):
        sl = i % 2
        wait_gather2(sl)
        if i + 2 < NCH_R:
            load_fidx(i + 2, sl)
        if i + 1 < NCH_R:
            if i >= 1:
                store_wait(i - 1, 1 - sl)
            wait_fidx(1 - sl)
            gather2(1 - sl)

        @plsc.parallel_loop(0, RK, unroll=2)
        def _(r):
            for j in range(HALF // 16):
                v = obuf[sl, r, pl.ds(j * 16, 16)]
                obuf[sl, r, pl.ds(j * 16, 16)] = jnp.maximum(v, v * 0.01)

        store2(i, sl)
    store_wait(NCH_R - 2, 0)
    store_wait(NCH_R - 1, 1)


def kernel(node_x, edge_index, features_n_f, W, b):
    flat = node_x.reshape(-1).astype(jnp.int32)
    h0, h1, g0, g1 = _tc_matmul(features_n_f, W, b)
    out = _sc_aggregate(edge_index[0].astype(jnp.int32),
                        edge_index[1].astype(jnp.int32), h0, h1, g0, g1,
                        flat)
    return out.reshape(node_x.shape[0], node_x.shape[1], HIDDEN)


# speedup vs baseline: 1.0613x; 1.0293x over previous
import functools

import jax
import jax.numpy as jnp
from jax import lax
from jax.experimental import pallas as pl
from jax.experimental.pallas import tpu as pltpu
from jax.experimental.pallas import tpu_sc as plsc

N_NODES = 10000
N_EDGES = 160000
D_FEAT = 256
HIDDEN = 256
HALF = 128
N_SEL = 10240

N_TILES = 16
EPT = N_EDGES // N_TILES
EK = 64
NCH_E = EPT // EK
ETAIL = EPT - NCH_E * EK
NG = 4
NI = 6
UNROLL = 12
RPT = N_SEL // N_TILES
RK = 32
NCH_R = RPT // RK
INIT_ROWS = 624
INIT_TAIL = N_NODES - N_TILES * INIT_ROWS
MTILE = 1000


def _mm_body(x_ref, w_ref, b_ref, o0_ref, o1_ref, o2_ref, o3_ref):
    x = x_ref[...]
    dn = (((1,), (1,)), ((), ()))
    h = lax.dot_general(x, w_ref[:, :D_FEAT], dn,
                        preferred_element_type=jnp.float32) + b_ref[...]
    g = lax.dot_general(x, w_ref[:, D_FEAT:], dn,
                        preferred_element_type=jnp.float32)
    o0_ref[...] = h[:, :HALF]
    o1_ref[...] = h[:, HALF:]
    o2_ref[...] = g[:, :HALF]
    o3_ref[...] = g[:, HALF:]


def _tc_matmul(features, w, b):
    part = pl.BlockSpec((MTILE, HALF), lambda m: (m, 0))
    pshape = jax.ShapeDtypeStruct((N_NODES, HALF), jnp.float32)
    return pl.pallas_call(
        _mm_body,
        grid=(N_NODES // MTILE,),
        in_specs=[
            pl.BlockSpec((MTILE, D_FEAT), lambda m: (m, 0)),
            pl.BlockSpec((HIDDEN, 2 * D_FEAT), lambda m: (0, 0)),
            pl.BlockSpec((1, HIDDEN), lambda m: (0, 0)),
        ],
        out_specs=[part, part, part, part],
        out_shape=[pshape, pshape, pshape, pshape],
    )(features, w, b.reshape(1, HIDDEN))


_SC_MESH = plsc.VectorSubcoreMesh(core_axis_name="c", subcore_axis_name="s")


@functools.partial(
    pl.kernel,
    out_type=jax.ShapeDtypeStruct((N_SEL, HIDDEN), jnp.float32),
    mesh=_SC_MESH,
    scratch_types=[
        pltpu.VMEM_SHARED((N_NODES, HALF), jnp.float32),
        *([pltpu.VMEM((EK,), jnp.int32)] * NI),
        *([pltpu.VMEM((EK,), jnp.int32)] * NI),
        pltpu.VMEM((ETAIL,), jnp.int32),
        pltpu.VMEM((ETAIL,), jnp.int32),
        pltpu.VMEM((NG, EK, HALF), jnp.float32),
        pltpu.VMEM((RK,), jnp.int32),
        pltpu.VMEM((RK,), jnp.int32),
        pltpu.VMEM((2, RK, HALF), jnp.float32),
        pltpu.SemaphoreType.DMA((NI,)),
        pltpu.SemaphoreType.DMA((NI,)),
        pltpu.SemaphoreType.DMA((NG,)),
        pltpu.SemaphoreType.DMA((NG,)),
        pltpu.SemaphoreType.DMA((2,)),
        pltpu.SemaphoreType.DMA((2,)),
        pltpu.SemaphoreType.DMA((2,)),
    ],
)
def _sc_aggregate(edges, h0, h1, g0, g1, flat, out, acc,
                  dx0, dx1, dx2, dx3, dx4, dx5,
                  sx0, sx1, sx2, sx3, sx4, sx5, dtail,
                  stail, grows, fx0, fx1, obuf, idsem, issem, gsem,
                  ssem, fsem, g2sem, osem):
    cc = lax.axis_index("c")
    s = lax.axis_index("s")
    dix = (dx0, dx1, dx2, dx3, dx4, dx5)
    six = (sx0, sx1, sx2, sx3, sx4, sx5)
    fx = (fx0, fx1)
    ebase = s * EPT

    def load_eidx(c, r):
        pltpu.async_copy(edges.at[pl.ds(ebase + c * EK, EK)], dix[r],
                         idsem.at[r])
        pltpu.async_copy(edges.at[pl.ds(N_EDGES + ebase + c * EK, EK)],
                         six[r], issem.at[r])

    def wait_eidx(r):
        pltpu.make_async_copy(edges.at[pl.ds(0, EK)], dix[r],
                              idsem.at[r]).wait()
        pltpu.make_async_copy(edges.at[pl.ds(0, EK)], six[r],
                              issem.at[r]).wait()

    def gather(r, q):
        @pl.when(cc == 0)
        def _():
            pltpu.async_copy(g0.at[six[r]], grows.at[q], gsem.at[q])

        @pl.when(cc == 1)
        def _():
            pltpu.async_copy(g1.at[six[r]], grows.at[q], gsem.at[q])

    def wait_gather(r, q):
        pltpu.make_async_copy(g0.at[six[r]], grows.at[q],
                              gsem.at[q]).wait()

    def scatter(r, q):
        pltpu.async_copy(grows.at[q], acc.at[dix[r]], ssem.at[q], add=True)

    def wait_scatter(r, q):
        pltpu.make_async_copy(grows.at[q], acc.at[dix[r]],
                              ssem.at[q]).wait()

    for k in range(NI):
        load_eidx(k, k)
    pltpu.async_copy(flat.at[pl.ds(s * RPT, RK)], fx0, fsem.at[0])

    r0 = s * INIT_ROWS

    @pl.when(cc == 0)
    def _():
        pltpu.sync_copy(h0.at[pl.ds(r0, INIT_ROWS)],
                        acc.at[pl.ds(r0, INIT_ROWS)])

    @pl.when(cc == 1)
    def _():
        pltpu.sync_copy(h1.at[pl.ds(r0, INIT_ROWS)],
                        acc.at[pl.ds(r0, INIT_ROWS)])

    tail0 = N_TILES * INIT_ROWS

    @pl.when(jnp.logical_and(s == N_TILES - 1, cc == 0))
    def _():
        pltpu.sync_copy(h0.at[pl.ds(tail0, INIT_TAIL)],
                        acc.at[pl.ds(tail0, INIT_TAIL)])

    @pl.when(jnp.logical_and(s == N_TILES - 1, cc == 1))
    def _():
        pltpu.sync_copy(h1.at[pl.ds(tail0, INIT_TAIL)],
                        acc.at[pl.ds(tail0, INIT_TAIL)])

    for k in range(3):
        wait_eidx(k)
        gather(k, k)
    plsc.subcore_barrier()

    def block(t, carry):
        for k in range(UNROLL):
            c = t * UNROLL + k
            q, r = k % NG, k % NI
            wait_gather(r, q)
            scatter(r, q)

            @pl.when(c >= 1)
            def _():
                wait_scatter((k - 1) % NI, (k - 1) % NG)

            @pl.when(jnp.logical_and(c >= 1, c + 5 < NCH_E))
            def _():
                load_eidx(c + 5, (k + 5) % NI)

            @pl.when(c + 3 < NCH_E)
            def _():
                wait_eidx((k + 3) % NI)
                gather((k + 3) % NI, (k + 3) % NG)

        return carry

    lax.fori_loop(0, NCH_E // UNROLL, block, 0)
    wait_scatter((NCH_E - 1) % NI, (NCH_E - 1) % NG)

    pltpu.sync_copy(edges.at[pl.ds(ebase + NCH_E * EK, ETAIL)], dtail)
    pltpu.sync_copy(edges.at[pl.ds(N_EDGES + ebase + NCH_E * EK, ETAIL)],
                    stail)
    tgt = grows.at[0, pl.ds(0, ETAIL)]

    @pl.when(cc == 0)
    def _():
        pltpu.async_copy(g0.at[stail], tgt, gsem.at[0]).wait()

    @pl.when(cc == 1)
    def _():
        pltpu.async_copy(g1.at[stail], tgt, gsem.at[0]).wait()

    pltpu.sync_copy(tgt, acc.at[dtail], add=True)
    plsc.subcore_barrier()

    fbase = s * RPT

    def load_fidx(i, sl):
        pltpu.async_copy(flat.at[pl.ds(fbase + i * RK, RK)], fx[sl],
                         fsem.at[sl])

    def wait_fidx(sl):
        pltpu.make_async_copy(flat.at[pl.ds(0, RK)], fx[sl],
                              fsem.at[sl]).wait()

    def gather2(sl):
        pltpu.async_copy(acc.at[fx[sl]], obuf.at[sl], g2sem.at[sl])

    def wait_gather2(sl):
        pltpu.make_async_copy(acc.at[fx[sl]], obuf.at[sl],
                              g2sem.at[sl]).wait()

    def store2(i, sl):
        b0 = fbase + i * RK

        @pl.when(cc == 0)
        def _():
            pltpu.async_copy(obuf.at[sl],
                             out.at[pl.ds(b0, RK), pl.ds(0, HALF)],
                             osem.at[sl])

        @pl.when(cc == 1)
        def _():
            pltpu.async_copy(obuf.at[sl],
                             out.at[pl.ds(b0, RK), pl.ds(HALF, HALF)],
                             osem.at[sl])

    def store_wait(i, sl):
        b0 = fbase + i * RK
        pltpu.make_async_copy(obuf.at[sl],
                              out.at[pl.ds(b0, RK), pl.ds(0, HALF)],
                              osem.at[sl]).wait()

    wait_fidx(0)
    load_fidx(1, 1)
    gather2(0)
    for i in range(NCH_R):
        sl = i % 2
        wait_gather2(sl)
        if i + 2 < NCH_R:
            load_fidx(i + 2, sl)
        if i + 1 < NCH_R:
            if i >= 1:
                store_wait(i - 1, 1 - sl)
            wait_fidx(1 - sl)
            gather2(1 - sl)

        @plsc.parallel_loop(0, RK, unroll=2)
        def _(r):
            for j in range(HALF // 16):
                v = obuf[sl, r, pl.ds(j * 16, 16)]
                obuf[sl, r, pl.ds(j * 16, 16)] = jnp.maximum(v, v * 0.01)

        store2(i, sl)
    store_wait(NCH_R - 2, 0)
    store_wait(NCH_R - 1, 1)


def kernel(node_x, edge_index, features_n_f, W, b):
    flat = node_x.reshape(-1).astype(jnp.int32)
    h0, h1, g0, g1 = _tc_matmul(features_n_f, W, b)
    out = _sc_aggregate(edge_index.reshape(-1).astype(jnp.int32),
                        h0, h1, g0, g1, flat)
    return out.reshape(node_x.shape[0], node_x.shape[1], HIDDEN)


# speedup vs baseline: 1.0816x; 1.0192x over previous
import functools

import jax
import jax.numpy as jnp
from jax import lax
from jax.experimental import pallas as pl
from jax.experimental.pallas import tpu as pltpu
from jax.experimental.pallas import tpu_sc as plsc

N_NODES = 10000
N_EDGES = 160000
D_FEAT = 256
HIDDEN = 256
HALF = 128
N_SEL = 10240

N_TILES = 16
EPT = N_EDGES // N_TILES
EK = 64
NCH_E = EPT // EK
ETAIL = EPT - NCH_E * EK
NG = 4
NI = 6
UNROLL = 12
RPT = N_SEL // N_TILES
RK = 32
NCH_R = RPT // RK
INIT_ROWS = 624
INIT_TAIL = N_NODES - N_TILES * INIT_ROWS
MTILE = 2000


def _mm_body(x_ref, w_ref, b_ref, o0_ref, o1_ref, o2_ref, o3_ref):
    x = x_ref[...]
    dn = (((1,), (1,)), ((), ()))
    h = lax.dot_general(x, w_ref[:, :D_FEAT], dn,
                        preferred_element_type=jnp.float32) + b_ref[...]
    g = lax.dot_general(x, w_ref[:, D_FEAT:], dn,
                        preferred_element_type=jnp.float32)
    o0_ref[...] = h[:, :HALF]
    o1_ref[...] = h[:, HALF:]
    o2_ref[...] = g[:, :HALF]
    o3_ref[...] = g[:, HALF:]


def _tc_matmul(features, w, b):
    part = pl.BlockSpec((MTILE, HALF), lambda m: (m, 0))
    pshape = jax.ShapeDtypeStruct((N_NODES, HALF), jnp.float32)
    return pl.pallas_call(
        _mm_body,
        grid=(N_NODES // MTILE,),
        in_specs=[
            pl.BlockSpec((MTILE, D_FEAT), lambda m: (m, 0)),
            pl.BlockSpec((HIDDEN, 2 * D_FEAT), lambda m: (0, 0)),
            pl.BlockSpec((1, HIDDEN), lambda m: (0, 0)),
        ],
        out_specs=[part, part, part, part],
        out_shape=[pshape, pshape, pshape, pshape],
    )(features, w, b.reshape(1, HIDDEN))


_SC_MESH = plsc.VectorSubcoreMesh(core_axis_name="c", subcore_axis_name="s")


@functools.partial(
    pl.kernel,
    out_type=jax.ShapeDtypeStruct((N_SEL, HIDDEN), jnp.float32),
    mesh=_SC_MESH,
    scratch_types=[
        pltpu.VMEM_SHARED((N_NODES, HALF), jnp.float32),
        *([pltpu.VMEM((EK,), jnp.int32)] * NI),
        *([pltpu.VMEM((EK,), jnp.int32)] * NI),
        pltpu.VMEM((ETAIL,), jnp.int32),
        pltpu.VMEM((ETAIL,), jnp.int32),
        pltpu.VMEM((NG, EK, HALF), jnp.float32),
        pltpu.VMEM((RK,), jnp.int32),
        pltpu.VMEM((RK,), jnp.int32),
        pltpu.VMEM((2, RK, HALF), jnp.float32),
        pltpu.SemaphoreType.DMA((NI,)),
        pltpu.SemaphoreType.DMA((NI,)),
        pltpu.SemaphoreType.DMA((NG,)),
        pltpu.SemaphoreType.DMA((NG,)),
        pltpu.SemaphoreType.DMA((2,)),
        pltpu.SemaphoreType.DMA((2,)),
        pltpu.SemaphoreType.DMA((2,)),
    ],
)
def _sc_aggregate(edges, h0, h1, g0, g1, flat, out, acc,
                  dx0, dx1, dx2, dx3, dx4, dx5,
                  sx0, sx1, sx2, sx3, sx4, sx5, dtail,
                  stail, grows, fx0, fx1, obuf, idsem, issem, gsem,
                  ssem, fsem, g2sem, osem):
    cc = lax.axis_index("c")
    s = lax.axis_index("s")
    dix = (dx0, dx1, dx2, dx3, dx4, dx5)
    six = (sx0, sx1, sx2, sx3, sx4, sx5)
    fx = (fx0, fx1)
    ebase = s * EPT

    def load_eidx(c, r):
        pltpu.async_copy(edges.at[pl.ds(ebase + c * EK, EK)], dix[r],
                         idsem.at[r])
        pltpu.async_copy(edges.at[pl.ds(N_EDGES + ebase + c * EK, EK)],
                         six[r], issem.at[r])

    def wait_eidx(r):
        pltpu.make_async_copy(edges.at[pl.ds(0, EK)], dix[r],
                              idsem.at[r]).wait()
        pltpu.make_async_copy(edges.at[pl.ds(0, EK)], six[r],
                              issem.at[r]).wait()

    def gather(r, q):
        @pl.when(cc == 0)
        def _():
            pltpu.async_copy(g0.at[six[r]], grows.at[q], gsem.at[q])

        @pl.when(cc == 1)
        def _():
            pltpu.async_copy(g1.at[six[r]], grows.at[q], gsem.at[q])

    def wait_gather(r, q):
        pltpu.make_async_copy(g0.at[six[r]], grows.at[q],
                              gsem.at[q]).wait()

    def scatter(r, q):
        pltpu.async_copy(grows.at[q], acc.at[dix[r]], ssem.at[q], add=True)

    def wait_scatter(r, q):
        pltpu.make_async_copy(grows.at[q], acc.at[dix[r]],
                              ssem.at[q]).wait()

    for k in range(NI):
        load_eidx(k, k)
    pltpu.async_copy(flat.at[pl.ds(s * RPT, RK)], fx0, fsem.at[0])

    r0 = s * INIT_ROWS

    @pl.when(cc == 0)
    def _():
        pltpu.sync_copy(h0.at[pl.ds(r0, INIT_ROWS)],
                        acc.at[pl.ds(r0, INIT_ROWS)])

    @pl.when(cc == 1)
    def _():
        pltpu.sync_copy(h1.at[pl.ds(r0, INIT_ROWS)],
                        acc.at[pl.ds(r0, INIT_ROWS)])

    tail0 = N_TILES * INIT_ROWS

    @pl.when(jnp.logical_and(s == N_TILES - 1, cc == 0))
    def _():
        pltpu.sync_copy(h0.at[pl.ds(tail0, INIT_TAIL)],
                        acc.at[pl.ds(tail0, INIT_TAIL)])

    @pl.when(jnp.logical_and(s == N_TILES - 1, cc == 1))
    def _():
        pltpu.sync_copy(h1.at[pl.ds(tail0, INIT_TAIL)],
                        acc.at[pl.ds(tail0, INIT_TAIL)])

    for k in range(3):
        wait_eidx(k)
        gather(k, k)
    plsc.subcore_barrier()

    def block(t, carry):
        for k in range(UNROLL):
            c = t * UNROLL + k
            q, r = k % NG, k % NI
            wait_gather(r, q)
            scatter(r, q)

            @pl.when(c >= 1)
            def _():
                wait_scatter((k - 1) % NI, (k - 1) % NG)

            @pl.when(jnp.logical_and(c >= 1, c + 5 < NCH_E))
            def _():
                load_eidx(c + 5, (k + 5) % NI)

            @pl.when(c + 3 < NCH_E)
            def _():
                wait_eidx((k + 3) % NI)
                gather((k + 3) % NI, (k + 3) % NG)

        return carry

    lax.fori_loop(0, NCH_E // UNROLL, block, 0)
    wait_scatter((NCH_E - 1) % NI, (NCH_E - 1) % NG)

    pltpu.sync_copy(edges.at[pl.ds(ebase + NCH_E * EK, ETAIL)], dtail)
    pltpu.sync_copy(edges.at[pl.ds(N_EDGES + ebase + NCH_E * EK, ETAIL)],
                    stail)
    tgt = grows.at[0, pl.ds(0, ETAIL)]

    @pl.when(cc == 0)
    def _():
        pltpu.async_copy(g0.at[stail], tgt, gsem.at[0]).wait()

    @pl.when(cc == 1)
    def _():
        pltpu.async_copy(g1.at[stail], tgt, gsem.at[0]).wait()

    pltpu.sync_copy(tgt, acc.at[dtail], add=True)
    plsc.subcore_barrier()

    fbase = s * RPT

    def load_fidx(i, sl):
        pltpu.async_copy(flat.at[pl.ds(fbase + i * RK, RK)], fx[sl],
                         fsem.at[sl])

    def wait_fidx(sl):
        pltpu.make_async_copy(flat.at[pl.ds(0, RK)], fx[sl],
                              fsem.at[sl]).wait()

    def gather2(sl):
        pltpu.async_copy(acc.at[fx[sl]], obuf.at[sl], g2sem.at[sl])

    def wait_gather2(sl):
        pltpu.make_async_copy(acc.at[fx[sl]], obuf.at[sl],
                              g2sem.at[sl]).wait()

    def store2(i, sl):
        b0 = fbase + i * RK

        @pl.when(cc == 0)
        def _():
            pltpu.async_copy(obuf.at[sl],
                             out.at[pl.ds(b0, RK), pl.ds(0, HALF)],
                             osem.at[sl])

        @pl.when(cc == 1)
        def _():
            pltpu.async_copy(obuf.at[sl],
                             out.at[pl.ds(b0, RK), pl.ds(HALF, HALF)],
                             osem.at[sl])

    def store_wait(i, sl):
        b0 = fbase + i * RK
        pltpu.make_async_copy(obuf.at[sl],
                              out.at[pl.ds(b0, RK), pl.ds(0, HALF)],
                              osem.at[sl]).wait()

    wait_fidx(0)
    load_fidx(1, 1)
    gather2(0)
    for i in range(NCH_R):
        sl = i % 2
        wait_gather2(sl)
        if i + 2 < NCH_R:
            load_fidx(i + 2, sl)
        if i + 1 < NCH_R:
            if i >= 1:
                store_wait(i - 1, 1 - sl)
            wait_fidx(1 - sl)
            gather2(1 - sl)

        @plsc.parallel_loop(0, RK, unroll=2)
        def _(r):
            for j in range(HALF // 16):
                v = obuf[sl, r, pl.ds(j * 16, 16)]
                obuf[sl, r, pl.ds(j * 16, 16)] = jnp.maximum(v, v * 0.01)

        store2(i, sl)
    store_wait(NCH_R - 2, 0)
    store_wait(NCH_R - 1, 1)


def kernel(node_x, edge_index, features_n_f, W, b):
    flat = node_x.reshape(-1).astype(jnp.int32)
    h0, h1, g0, g1 = _tc_matmul(features_n_f, W, b)
    out = _sc_aggregate(edge_index.reshape(-1).astype(jnp.int32),
                        h0, h1, g0, g1, flat)
    return out.reshape(node_x.shape[0], node_x.shape[1], HIDDEN)
